# Initial kernel scaffold; baseline (speedup 1.0000x reference)
import functools

import jax
import jax.numpy as jnp
from jax import lax
from jax.experimental import pallas as pl
from jax.experimental.pallas import tpu as pltpu
from jax.experimental.pallas import tpu_sc as plsc

NC = 2
NS = 16
NW = NC * NS
K = 128


def _sc_mesh():
    return plsc.VectorSubcoreMesh(core_axis_name="c", subcore_axis_name="s")


def _make_deg_kernel(n_pad, ch):
    zr = n_pad // NS

    @functools.partial(
        pl.kernel,
        mesh=_sc_mesh(),
        out_type=jax.ShapeDtypeStruct((NC * n_pad, 16), jnp.float32),
        scratch_types=[
            pltpu.VMEM((ch, K), jnp.int32),
            pltpu.VMEM((K, 16), jnp.float32),
            pltpu.VMEM_SHARED((n_pad, 16), jnp.float32),
        ],
    )
    def deg_kernel(row_hbm, zeros16_hbm, ones_hbm, degp_hbm, idx_v, ones_v, acc_sh):
        c = lax.axis_index("c")
        s = lax.axis_index("s")
        wid = s * NC + c
        pltpu.sync_copy(row_hbm.at[wid], idx_v)
        pltpu.sync_copy(ones_hbm, ones_v)
        pltpu.sync_copy(zeros16_hbm.at[pl.ds(s * zr, zr)], acc_sh.at[pl.ds(s * zr, zr)])
        plsc.subcore_barrier()

        def body(j, carry):
            pltpu.sync_copy(ones_v, acc_sh.at[idx_v.at[j]], add=True)
            return carry

        lax.fori_loop(0, ch, body, 0)
        plsc.subcore_barrier()
        pltpu.sync_copy(
            acc_sh.at[pl.ds(s * zr, zr)],
            degp_hbm.at[pl.ds(c * n_pad + s * zr, zr)],
        )

    return deg_kernel


def _make_mp_kernel(n_pad, d, ch):
    zr = n_pad // NS

    @functools.partial(
        pl.kernel,
        mesh=_sc_mesh(),
        out_type=jax.ShapeDtypeStruct((NC * n_pad, d), jnp.float32),
        scratch_types=[
            pltpu.VMEM((ch, K), jnp.int32),
            pltpu.VMEM((ch, K), jnp.int32),
            pltpu.VMEM((K, d), jnp.float32),
            pltpu.VMEM_SHARED((n_pad, d), jnp.float32),
            pltpu.SemaphoreType.DMA,
        ],
    )
    def mp_kernel(row_hbm, col_hbm, y_hbm, zeros_hbm, outp_hbm,
                  ridx_v, cidx_v, buf_v, acc_sh, sem):
        c = lax.axis_index("c")
        s = lax.axis_index("s")
        wid = s * NC + c
        pltpu.sync_copy(row_hbm.at[wid], ridx_v)
        pltpu.sync_copy(col_hbm.at[wid], cidx_v)
        pltpu.sync_copy(zeros_hbm.at[pl.ds(s * zr, zr)], acc_sh.at[pl.ds(s * zr, zr)])
        plsc.subcore_barrier()

        def body(j, carry):
            pltpu.async_copy(y_hbm.at[cidx_v.at[j]], buf_v, sem).wait()
            pltpu.sync_copy(buf_v, acc_sh.at[ridx_v.at[j]], add=True)
            return carry

        lax.fori_loop(0, ch, body, 0)
        plsc.subcore_barrier()
        pltpu.sync_copy(
            acc_sh.at[pl.ds(s * zr, zr)],
            outp_hbm.at[pl.ds(c * n_pad + s * zr, zr)],
        )

    return mp_kernel


def _lin_body(x_ref, wt_ref, b_ref, degp_ref, y_ref):
    deg = degp_ref[0, :, 0] + degp_ref[1, :, 0]
    dis = jnp.where(deg > 0, lax.rsqrt(deg), 0.0)
    xl = jnp.dot(x_ref[...], wt_ref[...], preferred_element_type=jnp.float32)
    y_ref[...] = (xl + b_ref[...]) * dis[:, None]


def _fin_body(outp_ref, degp_ref, o_ref):
    deg = degp_ref[0, :, 0] + degp_ref[1, :, 0]
    dis = jnp.where(deg > 0, lax.rsqrt(deg), 0.0)
    o_ref[...] = (outp_ref[0] + outp_ref[1]) * dis[:, None]


@jax.jit
def kernel(x, edge_index, W, b):
    n, d_in = x.shape
    d_out = W.shape[0]
    e = edge_index.shape[1]

    n_pad = (n // 256 + 1) * 256
    n_trash = n_pad - n
    ch = -(-e // (NW * K))
    ch += ch % 2
    e_pad = NW * K * ch

    pad_ids = n + (jnp.arange(e_pad - e, dtype=jnp.int32) % n_trash)
    row_p = jnp.concatenate([edge_index[0], pad_ids]).reshape(NW, ch, K)
    col_p = jnp.concatenate([edge_index[1], pad_ids]).reshape(NW, ch, K)

    zeros16 = jnp.zeros((n_pad, 16), jnp.float32)
    ones16 = jnp.ones((K, 16), jnp.float32)
    zeros_d = jnp.zeros((n_pad, d_out), jnp.float32)

    degp = _make_deg_kernel(n_pad, ch)(row_p, zeros16, ones16)
    degp3 = degp.reshape(NC, n_pad, 16)

    x_pad = jnp.pad(x, ((0, n_trash), (0, 0)))
    blk = 1024
    grid = (n_pad // blk,)
    y_pad = pl.pallas_call(
        _lin_body,
        grid=grid,
        in_specs=[
            pl.BlockSpec((blk, d_in), lambda i: (i, 0)),
            pl.BlockSpec((d_in, d_out), lambda i: (0, 0)),
            pl.BlockSpec((1, d_out), lambda i: (0, 0)),
            pl.BlockSpec((NC, blk, 16), lambda i: (0, i, 0)),
        ],
        out_specs=pl.BlockSpec((blk, d_out), lambda i: (i, 0)),
        out_shape=jax.ShapeDtypeStruct((n_pad, d_out), jnp.float32),
    )(x_pad, W.T, b.reshape(1, d_out), degp3)

    outp = _make_mp_kernel(n_pad, d_out, ch)(row_p, col_p, y_pad, zeros_d)
    outp3 = outp.reshape(NC, n_pad, d_out)

    out_pad = pl.pallas_call(
        _fin_body,
        grid=grid,
        in_specs=[
            pl.BlockSpec((NC, blk, d_out), lambda i: (0, i, 0)),
            pl.BlockSpec((NC, blk, 16), lambda i: (0, i, 0)),
        ],
        out_specs=pl.BlockSpec((blk, d_out), lambda i: (i, 0)),
        out_shape=jax.ShapeDtypeStruct((n_pad, d_out), jnp.float32),
    )(outp3, degp3)

    return out_pad[:n]


# speedup vs baseline: 26.4896x; 26.4896x over previous
import functools

import jax
import jax.numpy as jnp
from jax import lax
from jax.experimental import pallas as pl
from jax.experimental.pallas import tpu as pltpu
from jax.experimental.pallas import tpu_sc as plsc

NC = 2
NS = 16
NW = NC * NS
K = 128


def _sc_mesh():
    return plsc.VectorSubcoreMesh(
        core_axis_name="c", subcore_axis_name="s", num_cores=NC, num_subcores=NS
    )


def _make_deg_kernel(n_pad, ch):
    zr = n_pad // NS

    @functools.partial(
        pl.kernel,
        mesh=_sc_mesh(),
        out_type=jax.ShapeDtypeStruct((NC * n_pad,), jnp.float32),
        scratch_types=[
            pltpu.VMEM((ch, K), jnp.int32),
            pltpu.VMEM((n_pad,), jnp.float32),
            pltpu.VMEM((NS, zr), jnp.float32),
            pltpu.VMEM((zr,), jnp.float32),
            pltpu.VMEM_SHARED((NS, n_pad), jnp.float32),
        ],
        compiler_params=pltpu.CompilerParams(needs_layout_passes=False),
    )
    def deg_kernel(row_hbm, zeros_hbm, degp_hbm, idx_v, hist_v, red_v, out_v, slab_sh):
        c = lax.axis_index("c")
        s = lax.axis_index("s")
        wid = s * NC + c
        pltpu.sync_copy(row_hbm.at[wid], idx_v)
        pltpu.sync_copy(zeros_hbm, hist_v)

        def body(j, carry):
            for t in range(K // 16):
                vec = idx_v[j, pl.ds(t * 16, 16)]
                counts, lmask = plsc.scan_count(vec)
                plsc.addupdate_scatter(
                    hist_v, [vec], counts.astype(jnp.float32), mask=lmask
                )
            return carry

        lax.fori_loop(0, ch, body, 0)

        pltpu.sync_copy(hist_v, slab_sh.at[s])
        plsc.subcore_barrier()
        pltpu.sync_copy(slab_sh.at[:, pl.ds(s * zr, zr)], red_v)

        def rbody(t, carry):
            acc = red_v[0, pl.ds(t * 16, 16)]
            for r in range(1, NS):
                acc = acc + red_v[r, pl.ds(t * 16, 16)]
            out_v[pl.ds(t * 16, 16)] = acc
            return carry

        lax.fori_loop(0, zr // 16, rbody, 0)
        pltpu.sync_copy(out_v, degp_hbm.at[pl.ds(c * n_pad + s * zr, zr)])

    return deg_kernel


def _make_mp_kernel(n_pad, d, ch):
    zr = n_pad // NS

    @functools.partial(
        pl.kernel,
        mesh=_sc_mesh(),
        out_type=jax.ShapeDtypeStruct((NC * n_pad, d), jnp.float32),
        scratch_types=[
            pltpu.VMEM((ch, K), jnp.int32),
            pltpu.VMEM((ch, K), jnp.int32),
            pltpu.VMEM((K, d), jnp.float32),
            pltpu.VMEM_SHARED((n_pad, d), jnp.float32),
            pltpu.SemaphoreType.DMA,
        ],
    )
    def mp_kernel(row_hbm, col_hbm, y_hbm, zeros_hbm, outp_hbm,
                  ridx_v, cidx_v, buf_v, acc_sh, sem):
        c = lax.axis_index("c")
        s = lax.axis_index("s")
        wid = s * NC + c
        pltpu.sync_copy(row_hbm.at[wid], ridx_v)
        pltpu.sync_copy(col_hbm.at[wid], cidx_v)
        pltpu.sync_copy(zeros_hbm.at[pl.ds(s * zr, zr)], acc_sh.at[pl.ds(s * zr, zr)])
        plsc.subcore_barrier()

        def body(j, carry):
            pltpu.async_copy(y_hbm.at[cidx_v.at[j]], buf_v, sem).wait()
            pltpu.sync_copy(buf_v, acc_sh.at[ridx_v.at[j]], add=True)
            return carry

        lax.fori_loop(0, ch, body, 0)
        plsc.subcore_barrier()
        pltpu.sync_copy(
            acc_sh.at[pl.ds(s * zr, zr)],
            outp_hbm.at[pl.ds(c * n_pad + s * zr, zr)],
        )

    return mp_kernel


def _lin_body(x_ref, wt_ref, b_ref, p0_ref, p1_ref, y_ref):
    deg = p0_ref[...] + p1_ref[...]
    dis = jnp.where(deg > 0, lax.rsqrt(deg), 0.0)
    xl = jnp.dot(x_ref[...], wt_ref[...], preferred_element_type=jnp.float32)
    y_ref[...] = (xl + b_ref[...]) * dis[:, None]


def _fin_body(outp_ref, p0_ref, p1_ref, o_ref):
    deg = p0_ref[...] + p1_ref[...]
    dis = jnp.where(deg > 0, lax.rsqrt(deg), 0.0)
    o_ref[...] = (outp_ref[0] + outp_ref[1]) * dis[:, None]


@jax.jit
def kernel(x, edge_index, W, b):
    n, d_in = x.shape
    d_out = W.shape[0]
    e = edge_index.shape[1]

    n_pad = (n // 256 + 1) * 256
    n_trash = n_pad - n
    ch = -(-e // (NW * K))
    ch += ch % 2
    e_pad = NW * K * ch

    pad_ids = n + (jnp.arange(e_pad - e, dtype=jnp.int32) % n_trash)
    row_p = jnp.concatenate([edge_index[0], pad_ids]).reshape(NW, ch, K)
    col_p = jnp.concatenate([edge_index[1], pad_ids]).reshape(NW, ch, K)

    zeros1 = jnp.zeros((n_pad,), jnp.float32)
    zeros_d = jnp.zeros((n_pad, d_out), jnp.float32)

    degp = _make_deg_kernel(n_pad, ch)(row_p, zeros1)
    p0 = degp[:n_pad]
    p1 = degp[n_pad:]

    x_pad = jnp.pad(x, ((0, n_trash), (0, 0)))
    blk = 1024
    grid = (n_pad // blk,)
    y_pad = pl.pallas_call(
        _lin_body,
        grid=grid,
        in_specs=[
            pl.BlockSpec((blk, d_in), lambda i: (i, 0)),
            pl.BlockSpec((d_in, d_out), lambda i: (0, 0)),
            pl.BlockSpec((1, d_out), lambda i: (0, 0)),
            pl.BlockSpec((blk,), lambda i: (i,)),
            pl.BlockSpec((blk,), lambda i: (i,)),
        ],
        out_specs=pl.BlockSpec((blk, d_out), lambda i: (i, 0)),
        out_shape=jax.ShapeDtypeStruct((n_pad, d_out), jnp.float32),
    )(x_pad, W.T, b.reshape(1, d_out), p0, p1)

    outp = _make_mp_kernel(n_pad, d_out, ch)(row_p, col_p, y_pad, zeros_d)
    outp3 = outp.reshape(NC, n_pad, d_out)

    out_pad = pl.pallas_call(
        _fin_body,
        grid=grid,
        in_specs=[
            pl.BlockSpec((NC, blk, d_out), lambda i: (0, i, 0)),
            pl.BlockSpec((blk,), lambda i: (i,)),
            pl.BlockSpec((blk,), lambda i: (i,)),
        ],
        out_specs=pl.BlockSpec((blk, d_out), lambda i: (i, 0)),
        out_shape=jax.ShapeDtypeStruct((n_pad, d_out), jnp.float32),
    )(outp3, p0, p1)

    return out_pad[:n]


# speedup vs baseline: 31.2147x; 1.1784x over previous
import functools

import jax
import jax.numpy as jnp
from jax import lax
from jax.experimental import pallas as pl
from jax.experimental.pallas import tpu as pltpu
from jax.experimental.pallas import tpu_sc as plsc

NC = 2
NS = 16
NW = NC * NS
K = 128


def _sc_mesh():
    return plsc.VectorSubcoreMesh(
        core_axis_name="c", subcore_axis_name="s", num_cores=NC, num_subcores=NS
    )


def _make_deg_kernel(n_pad, ch):
    zr = n_pad // NS

    @functools.partial(
        pl.kernel,
        mesh=_sc_mesh(),
        out_type=jax.ShapeDtypeStruct((NC * n_pad,), jnp.float32),
        scratch_types=[
            pltpu.VMEM((ch, K), jnp.int32),
            pltpu.VMEM((n_pad,), jnp.float32),
            pltpu.VMEM((NS, zr), jnp.float32),
            pltpu.VMEM((zr,), jnp.float32),
            pltpu.VMEM_SHARED((NS, n_pad), jnp.float32),
        ],
        compiler_params=pltpu.CompilerParams(needs_layout_passes=False),
    )
    def deg_kernel(row_hbm, zeros_hbm, degp_hbm, idx_v, hist_v, red_v, out_v, slab_sh):
        c = lax.axis_index("c")
        s = lax.axis_index("s")
        wid = s * NC + c
        pltpu.sync_copy(row_hbm.at[wid], idx_v)
        pltpu.sync_copy(zeros_hbm, hist_v)

        def body(j, carry):
            for t in range(K // 16):
                vec = idx_v[j, pl.ds(t * 16, 16)]
                counts, lmask = plsc.scan_count(vec)
                plsc.addupdate_scatter(
                    hist_v, [vec], counts.astype(jnp.float32), mask=lmask
                )
            return carry

        lax.fori_loop(0, ch, body, 0)

        pltpu.sync_copy(hist_v, slab_sh.at[s])
        plsc.subcore_barrier()
        pltpu.sync_copy(slab_sh.at[:, pl.ds(s * zr, zr)], red_v)

        def rbody(t, carry):
            acc = red_v[0, pl.ds(t * 16, 16)]
            for r in range(1, NS):
                acc = acc + red_v[r, pl.ds(t * 16, 16)]
            out_v[pl.ds(t * 16, 16)] = acc
            return carry

        lax.fori_loop(0, zr // 16, rbody, 0)
        pltpu.sync_copy(out_v, degp_hbm.at[pl.ds(c * n_pad + s * zr, zr)])

    return deg_kernel


def _make_mp_kernel(n_pad, d, ch, nbuf=2):
    zr = n_pad // NS
    assert ch % nbuf == 0

    @functools.partial(
        pl.kernel,
        mesh=_sc_mesh(),
        out_type=jax.ShapeDtypeStruct((NC * n_pad, d), jnp.float32),
        scratch_types=(
            [pltpu.VMEM((K,), jnp.int32) for _ in range(2 * nbuf)]
            + [pltpu.VMEM((K, d), jnp.float32) for _ in range(nbuf)]
            + [
                pltpu.VMEM_SHARED((n_pad, d), jnp.float32),
                pltpu.SemaphoreType.DMA((nbuf,)),
                pltpu.SemaphoreType.DMA((nbuf,)),
            ]
        ),
    )
    def mp_kernel(row_hbm, col_hbm, y_hbm, zeros_hbm, outp_hbm, *rest):
        ridx = rest[:nbuf]
        cidx = rest[nbuf:2 * nbuf]
        bufs = rest[2 * nbuf:3 * nbuf]
        acc_sh, gsems, isems = rest[3 * nbuf], rest[3 * nbuf + 1], rest[3 * nbuf + 2]
        c = lax.axis_index("c")
        s = lax.axis_index("s")
        wid = s * NC + c
        base = wid * ch
        pltpu.sync_copy(zeros_hbm.at[pl.ds(s * zr, zr)], acc_sh.at[pl.ds(s * zr, zr)])
        plsc.subcore_barrier()

        for b in range(nbuf):
            pltpu.sync_copy(row_hbm.at[base + b], ridx[b])
            pltpu.sync_copy(col_hbm.at[base + b], cidx[b])
            pltpu.async_copy(y_hbm.at[cidx[b]], bufs[b], gsems.at[b])

        def body(g, carry):
            for b in range(nbuf):
                j = g * nbuf + b
                pltpu.make_async_copy(
                    y_hbm.at[cidx[b]], bufs[b], gsems.at[b]
                ).wait()
                pltpu.sync_copy(bufs[b], acc_sh.at[ridx[b]], add=True)
                pltpu.async_copy(row_hbm.at[base + j + nbuf], ridx[b], isems.at[b])
                pltpu.async_copy(col_hbm.at[base + j + nbuf], cidx[b], isems.at[b])
            for b in range(nbuf):
                pltpu.make_async_copy(
                    row_hbm.at[base], ridx[b], isems.at[b]
                ).wait()
                pltpu.make_async_copy(
                    col_hbm.at[base], cidx[b], isems.at[b]
                ).wait()
                pltpu.async_copy(y_hbm.at[cidx[b]], bufs[b], gsems.at[b])
            return carry

        lax.fori_loop(0, ch // nbuf - 1, body, 0)
        for b in range(nbuf):
            pltpu.make_async_copy(
                y_hbm.at[cidx[b]], bufs[b], gsems.at[b]
            ).wait()
            pltpu.sync_copy(bufs[b], acc_sh.at[ridx[b]], add=True)
        plsc.subcore_barrier()
        pltpu.sync_copy(
            acc_sh.at[pl.ds(s * zr, zr)],
            outp_hbm.at[pl.ds(c * n_pad + s * zr, zr)],
        )

    return mp_kernel


def _lin_body(x_ref, wt_ref, b_ref, p0_ref, p1_ref, y_ref):
    deg = p0_ref[...] + p1_ref[...]
    dis = jnp.where(deg > 0, lax.rsqrt(deg), 0.0)
    xl = jnp.dot(x_ref[...], wt_ref[...], preferred_element_type=jnp.float32)
    y_ref[...] = (xl + b_ref[...]) * dis[:, None]


def _fin_body(outp_ref, p0_ref, p1_ref, o_ref):
    deg = p0_ref[...] + p1_ref[...]
    dis = jnp.where(deg > 0, lax.rsqrt(deg), 0.0)
    o_ref[...] = (outp_ref[0] + outp_ref[1]) * dis[:, None]


@jax.jit
def kernel(x, edge_index, W, b):
    n, d_in = x.shape
    d_out = W.shape[0]
    e = edge_index.shape[1]

    n_pad = (n // 256 + 1) * 256
    n_trash = n_pad - n
    ch = -(-e // (NW * K))
    ch = -(-ch // 2) * 2
    e_pad = NW * K * ch

    pad_ids = n + (jnp.arange(e_pad - e, dtype=jnp.int32) % n_trash)
    row_p = jnp.concatenate([edge_index[0], pad_ids]).reshape(NW, ch, K)
    col_p = jnp.concatenate([edge_index[1], pad_ids]).reshape(NW, ch, K)

    zeros1 = jnp.zeros((n_pad,), jnp.float32)
    zeros_d = jnp.zeros((n_pad, d_out), jnp.float32)

    degp = _make_deg_kernel(n_pad, ch)(row_p, zeros1)
    p0 = degp[:n_pad]
    p1 = degp[n_pad:]

    x_pad = jnp.pad(x, ((0, n_trash), (0, 0)))
    blk = 1024
    grid = (n_pad // blk,)
    y_pad = pl.pallas_call(
        _lin_body,
        grid=grid,
        in_specs=[
            pl.BlockSpec((blk, d_in), lambda i: (i, 0)),
            pl.BlockSpec((d_in, d_out), lambda i: (0, 0)),
            pl.BlockSpec((1, d_out), lambda i: (0, 0)),
            pl.BlockSpec((blk,), lambda i: (i,)),
            pl.BlockSpec((blk,), lambda i: (i,)),
        ],
        out_specs=pl.BlockSpec((blk, d_out), lambda i: (i, 0)),
        out_shape=jax.ShapeDtypeStruct((n_pad, d_out), jnp.float32),
    )(x_pad, W.T, b.reshape(1, d_out), p0, p1)

    outp = _make_mp_kernel(n_pad, d_out, ch)(
        row_p.reshape(NW * ch, K), col_p.reshape(NW * ch, K), y_pad, zeros_d
    )
    outp3 = outp.reshape(NC, n_pad, d_out)

    out_pad = pl.pallas_call(
        _fin_body,
        grid=grid,
        in_specs=[
            pl.BlockSpec((NC, blk, d_out), lambda i: (0, i, 0)),
            pl.BlockSpec((blk,), lambda i: (i,)),
            pl.BlockSpec((blk,), lambda i: (i,)),
        ],
        out_specs=pl.BlockSpec((blk, d_out), lambda i: (i, 0)),
        out_shape=jax.ShapeDtypeStruct((n_pad, d_out), jnp.float32),
    )(outp3, p0, p1)

    return out_pad[:n]
